# baseline (speedup 1.0000x reference)
import functools

import jax
import jax.numpy as jnp
from jax import lax
from jax.experimental import pallas as pl
from jax.experimental.pallas import tpu as pltpu
from jax.experimental.pallas import tpu_sc as plsc

V = 1000001
D = 64
WC_ROWS = 500032
CH = 512
NCH = 61
MAIN_END = 32 * NCH * CH
PITCH = 513


def _transpose_table(Wt, Wtail):
    mesh = plsc.VectorSubcoreMesh(core_axis_name="c", subcore_axis_name="s")

    @functools.partial(
        pl.kernel,
        mesh=mesh,
        compiler_params=pltpu.CompilerParams(
            use_tc_tiling_on_sc=True, needs_layout_passes=False
        ),
        out_type=jax.ShapeDtypeStruct((WC_ROWS, 128), jnp.float32),
        scratch_types=[
            pltpu.VMEM((D, PITCH), jnp.float32),
            pltpu.VMEM((CH // 2, 128), jnp.float32),
        ],
    )
    def k(wt_hbm, wtail_hbm, wc_hbm, inb, outb):
        wid = lax.axis_index("s") * 2 + lax.axis_index("c")
        lanes = lax.broadcasted_iota(jnp.int32, (16,), 0)
        dvecs = [d0 + lanes for d0 in (0, 16, 32, 48)]
        zeros = lanes * 0

        def do_chunk(v0, width):
            v0 = pl.multiple_of(v0, 128)
            pltpu.sync_copy(
                wt_hbm.at[:, pl.ds(v0, width)], inb.at[:, pl.ds(0, width)]
            )

            def row(v, carry):
                rvec = zeros + lax.shift_right_logical(v, 1)
                c0 = (v & 1) * 64
                col = zeros + v
                for i, dv in enumerate(dvecs):
                    vals = plsc.load_gather(inb, [dv, col])
                    plsc.store_scatter(
                        outb, [rvec, lanes + (c0 + i * 16)], vals
                    )
                return carry

            lax.fori_loop(0, width, row, 0, unroll=4)
            pltpu.sync_copy(
                outb.at[pl.ds(0, width // 2)],
                wc_hbm.at[pl.ds(pl.multiple_of(v0 // 2, 8), width // 2)],
            )

        def chunk_body(c, _):
            do_chunk(wid * (NCH * CH) + c * CH, CH)
            return _

        lax.fori_loop(0, NCH, chunk_body, 0)
        @pl.when(wid == 0)
        def _():
            do_chunk(MAIN_END, CH)

        @pl.when(wid == 1)
        def _():
            pltpu.sync_copy(wtail_hbm, outb.at[pl.ds(0, 32)])
            pltpu.sync_copy(
                outb.at[pl.ds(0, 32)],
                wc_hbm.at[pl.ds((MAIN_END + CH) // 2, 32)],
            )

    return k(Wt, Wtail)


@jax.jit
def _prep_table(W):
    Wtail = W[MAIN_END + CH : MAIN_END + CH + 64].reshape(32, 128)
    return _transpose_table(jnp.transpose(W), Wtail)


def _gather_blocks(Wc, inpT):
    mesh = plsc.VectorSubcoreMesh(core_axis_name="c", subcore_axis_name="s")

    @functools.partial(
        pl.kernel,
        mesh=mesh,
        compiler_params=pltpu.CompilerParams(
            use_tc_tiling_on_sc=True, needs_layout_passes=False
        ),
        out_type=jax.ShapeDtypeStruct((200, 8, 32, 8, 128), jnp.float32),
        scratch_types=[
            pltpu.VMEM((8, 128), jnp.int32),
            pltpu.VMEM((8, 128), jnp.int32),
            pltpu.VMEM((8, 128), jnp.int32),
            pltpu.VMEM((128, 128), jnp.float32),
            pltpu.VMEM((64, 129), jnp.float32),
            pltpu.SemaphoreType.DMA,
        ],
    )
    def k(wc_hbm, idx_hbm, out_hbm, idxb, idx2v, offv, gbuf, obuf, sem):
        wid = lax.axis_index("s") * 2 + lax.axis_index("c")
        bt = wid
        c0 = pl.multiple_of(bt * 128, 128)
        lanes = lax.broadcasted_iota(jnp.int32, (16,), 0)
        zeros = lanes * 0

        def unit(i, carry):
            h0 = pl.multiple_of(i * 8, 8)
            pltpu.sync_copy(idx_hbm.at[pl.ds(h0, 8), pl.ds(c0, 128)], idxb)
            for r in range(8):
                for kk in range(8):
                    v = idxb[r, pl.ds(kk * 16, 16)]
                    idx2v[r, pl.ds(kk * 16, 16)] = (
                        lax.shift_right_logical(v, 1)
                    )
                    offv[r, pl.ds(kk * 16, 16)] = (v & 1) * 64
            for r in range(8):
                pltpu.async_copy(wc_hbm.at[idx2v.at[r]], gbuf, sem).wait()

                def row(l, carry2):
                    lvec = zeros + l
                    off = plsc.load_gather(offv, [zeros + r, lvec])
                    for di in range(4):
                        vals = plsc.load_gather(
                            gbuf, [lvec, off + di * 16 + lanes]
                        )
                        plsc.store_scatter(
                            obuf, [di * 16 + lanes, lvec], vals
                        )
                    return carry2

                lax.fori_loop(0, 128, row, 0)
                for dt in range(8):
                    pltpu.sync_copy(
                        obuf.at[pl.ds(dt * 8, 8), pl.ds(0, 128)],
                        out_hbm.at[h0 + r, dt, bt],
                    )
            return carry

        lax.fori_loop(0, 25, unit, 0)

    return k(Wc, inpT)


def kernel(inp, W):
    Wc = _prep_table(W)
    out5 = _gather_blocks(Wc, jnp.transpose(inp).astype(jnp.int32))
    return out5.transpose(2, 4, 0, 1, 3).reshape(4096, 200, 64)


# speedup vs baseline: 1.4486x; 1.4486x over previous
import functools

import jax
import jax.numpy as jnp
from jax import lax
from jax.experimental import pallas as pl
from jax.experimental.pallas import tpu as pltpu
from jax.experimental.pallas import tpu_sc as plsc

V = 1000001
D = 64
WC_ROWS = 500032
CH = 512
NCH = 61
MAIN_END = 32 * NCH * CH
PITCH = 513


def _transpose_table(Wt, Wtail):
    mesh = plsc.VectorSubcoreMesh(core_axis_name="c", subcore_axis_name="s")

    @functools.partial(
        pl.kernel,
        mesh=mesh,
        compiler_params=pltpu.CompilerParams(
            use_tc_tiling_on_sc=True, needs_layout_passes=False
        ),
        out_type=jax.ShapeDtypeStruct((WC_ROWS, 128), jnp.float32),
        scratch_types=[
            pltpu.VMEM((D, PITCH), jnp.float32),
            pltpu.VMEM((CH // 2, 128), jnp.float32),
        ],
    )
    def k(wt_hbm, wtail_hbm, wc_hbm, inb, outb):
        wid = lax.axis_index("s") * 2 + lax.axis_index("c")
        lanes = lax.broadcasted_iota(jnp.int32, (16,), 0)
        dvecs = [d0 + lanes for d0 in (0, 16, 32, 48)]
        zeros = lanes * 0

        def do_chunk(v0, width):
            v0 = pl.multiple_of(v0, 128)
            pltpu.sync_copy(
                wt_hbm.at[:, pl.ds(v0, width)], inb.at[:, pl.ds(0, width)]
            )

            @plsc.parallel_loop(0, width, unroll=8)
            def row(v):
                rvec = zeros + lax.shift_right_logical(v, 1)
                c0 = (v & 1) * 64
                col = zeros + v
                for i, dv in enumerate(dvecs):
                    vals = plsc.load_gather(inb, [dv, col])
                    plsc.store_scatter(
                        outb, [rvec, lanes + (c0 + i * 16)], vals
                    )
            pltpu.sync_copy(
                outb.at[pl.ds(0, width // 2)],
                wc_hbm.at[pl.ds(pl.multiple_of(v0 // 2, 8), width // 2)],
            )

        def chunk_body(c, _):
            do_chunk(wid * (NCH * CH) + c * CH, CH)
            return _

        lax.fori_loop(0, NCH, chunk_body, 0)
        @pl.when(wid == 0)
        def _():
            do_chunk(MAIN_END, CH)

        @pl.when(wid == 1)
        def _():
            pltpu.sync_copy(wtail_hbm, outb.at[pl.ds(0, 32)])
            pltpu.sync_copy(
                outb.at[pl.ds(0, 32)],
                wc_hbm.at[pl.ds((MAIN_END + CH) // 2, 32)],
            )

    return k(Wt, Wtail)


@jax.jit
def _prep_table(W):
    Wtail = W[MAIN_END + CH : MAIN_END + CH + 64].reshape(32, 128)
    return _transpose_table(jnp.transpose(W), Wtail)


def _gather_blocks(Wc, inpT):
    mesh = plsc.VectorSubcoreMesh(core_axis_name="c", subcore_axis_name="s")

    @functools.partial(
        pl.kernel,
        mesh=mesh,
        compiler_params=pltpu.CompilerParams(
            use_tc_tiling_on_sc=True, needs_layout_passes=False
        ),
        out_type=jax.ShapeDtypeStruct((200, 8, 32, 8, 128), jnp.float32),
        scratch_types=[
            pltpu.VMEM((8, 128), jnp.int32),
            pltpu.VMEM((8, 128), jnp.int32),
            pltpu.VMEM((8, 128), jnp.int32),
            pltpu.VMEM((128, 128), jnp.float32),
            pltpu.VMEM((64, 129), jnp.float32),
            pltpu.SemaphoreType.DMA,
        ],
    )
    def k(wc_hbm, idx_hbm, out_hbm, idxb, idx2v, offv, gbuf, obuf, sem):
        wid = lax.axis_index("s") * 2 + lax.axis_index("c")
        bt = wid
        c0 = pl.multiple_of(bt * 128, 128)
        lanes = lax.broadcasted_iota(jnp.int32, (16,), 0)
        zeros = lanes * 0

        def unit(i, carry):
            h0 = pl.multiple_of(i * 8, 8)
            pltpu.sync_copy(idx_hbm.at[pl.ds(h0, 8), pl.ds(c0, 128)], idxb)
            for r in range(8):
                for kk in range(8):
                    v = idxb[r, pl.ds(kk * 16, 16)]
                    idx2v[r, pl.ds(kk * 16, 16)] = (
                        lax.shift_right_logical(v, 1)
                    )
                    offv[r, pl.ds(kk * 16, 16)] = (v & 1) * 64
            for r in range(8):
                pltpu.async_copy(wc_hbm.at[idx2v.at[r]], gbuf, sem).wait()

                @plsc.parallel_loop(0, 128, unroll=8)
                def row(l):
                    lvec = zeros + l
                    off = plsc.load_gather(offv, [zeros + r, lvec])
                    for di in range(4):
                        vals = plsc.load_gather(
                            gbuf, [lvec, off + di * 16 + lanes]
                        )
                        plsc.store_scatter(
                            obuf, [di * 16 + lanes, lvec], vals
                        )
                for dt in range(8):
                    pltpu.sync_copy(
                        obuf.at[pl.ds(dt * 8, 8), pl.ds(0, 128)],
                        out_hbm.at[h0 + r, dt, bt],
                    )
            return carry

        lax.fori_loop(0, 25, unit, 0)

    return k(Wc, inpT)


def kernel(inp, W):
    Wc = _prep_table(W)
    out5 = _gather_blocks(Wc, jnp.transpose(inp).astype(jnp.int32))
    return out5.transpose(2, 4, 0, 1, 3).reshape(4096, 200, 64)


# speedup vs baseline: 1.4570x; 1.0058x over previous
import functools

import jax
import jax.numpy as jnp
from jax import lax
from jax.experimental import pallas as pl
from jax.experimental.pallas import tpu as pltpu
from jax.experimental.pallas import tpu_sc as plsc

V = 1000001
D = 64
WC_ROWS = 500032
CH = 512
NCH = 61
MAIN_END = 32 * NCH * CH
PITCH = 513


def _transpose_table(Wt, Wtail):
    mesh = plsc.VectorSubcoreMesh(core_axis_name="c", subcore_axis_name="s")

    @functools.partial(
        pl.kernel,
        mesh=mesh,
        compiler_params=pltpu.CompilerParams(
            use_tc_tiling_on_sc=True, needs_layout_passes=False
        ),
        out_type=jax.ShapeDtypeStruct((WC_ROWS, 128), jnp.float32),
        scratch_types=[
            pltpu.VMEM((D, PITCH), jnp.float32),
            pltpu.VMEM((CH // 2, 128), jnp.float32),
        ],
    )
    def k(wt_hbm, wtail_hbm, wc_hbm, inb, outb):
        wid = lax.axis_index("s") * 2 + lax.axis_index("c")
        lanes = lax.broadcasted_iota(jnp.int32, (16,), 0)
        dvecs = [d0 + lanes for d0 in (0, 16, 32, 48)]
        zeros = lanes * 0

        def do_chunk(v0, width):
            v0 = pl.multiple_of(v0, 128)
            pltpu.sync_copy(
                wt_hbm.at[:, pl.ds(v0, width)], inb.at[:, pl.ds(0, width)]
            )

            @plsc.parallel_loop(0, width // 2, unroll=8)
            def row(u):
                ce = zeros + 2 * u
                co = ce + 1
                for i, dv in enumerate(dvecs):
                    outb[u, pl.ds(i * 16, 16)] = plsc.load_gather(
                        inb, [dv, ce]
                    )
                    outb[u, pl.ds(64 + i * 16, 16)] = plsc.load_gather(
                        inb, [dv, co]
                    )
            pltpu.sync_copy(
                outb.at[pl.ds(0, width // 2)],
                wc_hbm.at[pl.ds(pl.multiple_of(v0 // 2, 8), width // 2)],
            )

        def chunk_body(c, _):
            do_chunk(wid * (NCH * CH) + c * CH, CH)
            return _

        lax.fori_loop(0, NCH, chunk_body, 0)
        @pl.when(wid == 0)
        def _():
            do_chunk(MAIN_END, CH)

        @pl.when(wid == 1)
        def _():
            pltpu.sync_copy(wtail_hbm, outb.at[pl.ds(0, 32)])
            pltpu.sync_copy(
                outb.at[pl.ds(0, 32)],
                wc_hbm.at[pl.ds((MAIN_END + CH) // 2, 32)],
            )

    return k(Wt, Wtail)


@jax.jit
def _prep_table(W):
    Wtail = W[MAIN_END + CH : MAIN_END + CH + 64].reshape(32, 128)
    return _transpose_table(jnp.transpose(W), Wtail)


def _gather_blocks(Wc, inpT):
    mesh = plsc.VectorSubcoreMesh(core_axis_name="c", subcore_axis_name="s")

    @functools.partial(
        pl.kernel,
        mesh=mesh,
        compiler_params=pltpu.CompilerParams(
            use_tc_tiling_on_sc=True, needs_layout_passes=False
        ),
        out_type=jax.ShapeDtypeStruct((200, 8, 32, 8, 128), jnp.float32),
        scratch_types=[
            pltpu.VMEM((8, 128), jnp.int32),
            pltpu.VMEM((8, 128), jnp.int32),
            pltpu.VMEM((8, 128), jnp.int32),
            pltpu.VMEM((128, 128), jnp.float32),
            pltpu.VMEM((64, 129), jnp.float32),
            pltpu.SemaphoreType.DMA,
        ],
    )
    def k(wc_hbm, idx_hbm, out_hbm, idxb, idx2v, offv, gbuf, obuf, sem):
        wid = lax.axis_index("s") * 2 + lax.axis_index("c")
        bt = wid
        c0 = pl.multiple_of(bt * 128, 128)
        lanes = lax.broadcasted_iota(jnp.int32, (16,), 0)
        zeros = lanes * 0

        def unit(i, carry):
            h0 = pl.multiple_of(i * 8, 8)
            pltpu.sync_copy(idx_hbm.at[pl.ds(h0, 8), pl.ds(c0, 128)], idxb)
            for r in range(8):
                for kk in range(8):
                    v = idxb[r, pl.ds(kk * 16, 16)]
                    idx2v[r, pl.ds(kk * 16, 16)] = (
                        lax.shift_right_logical(v, 1)
                    )
                    offv[r, pl.ds(kk * 16, 16)] = (v & 1) * 64
            for r in range(8):
                pltpu.async_copy(wc_hbm.at[idx2v.at[r]], gbuf, sem).wait()

                @plsc.parallel_loop(0, 128, unroll=8)
                def row(l):
                    lvec = zeros + l
                    off = plsc.load_gather(offv, [zeros + r, lvec])
                    for di in range(4):
                        vals = plsc.load_gather(
                            gbuf, [lvec, off + di * 16 + lanes]
                        )
                        plsc.store_scatter(
                            obuf, [di * 16 + lanes, lvec], vals
                        )
                for dt in range(8):
                    pltpu.sync_copy(
                        obuf.at[pl.ds(dt * 8, 8), pl.ds(0, 128)],
                        out_hbm.at[h0 + r, dt, bt],
                    )
            return carry

        lax.fori_loop(0, 25, unit, 0)

    return k(Wc, inpT)


def kernel(inp, W):
    Wc = _prep_table(W)
    out5 = _gather_blocks(Wc, jnp.transpose(inp).astype(jnp.int32))
    return out5.transpose(2, 4, 0, 1, 3).reshape(4096, 200, 64)


# speedup vs baseline: 2.7132x; 1.8622x over previous
import functools

import jax
import jax.numpy as jnp
from jax import lax
from jax.experimental import pallas as pl
from jax.experimental.pallas import tpu as pltpu
from jax.experimental.pallas import tpu_sc as plsc

_CHUNK = 800


@functools.partial(jax.jit, static_argnames=("B", "D"))
def _gather_rows(idx_flat, W, B, D):
    info = plsc.get_sparse_core_info()
    NC, NS = info.num_cores, info.num_subcores
    NW = NC * NS
    b_per_w = B // NW
    n_chunks = b_per_w // _CHUNK
    C = _CHUNK
    mesh = plsc.VectorSubcoreMesh(core_axis_name="c", subcore_axis_name="s")

    @functools.partial(
        pl.kernel,
        mesh=mesh,
        compiler_params=pltpu.CompilerParams(use_tc_tiling_on_sc=False),
        out_type=jax.ShapeDtypeStruct((B, D), jnp.float32),
        scratch_types=[
            pltpu.VMEM((b_per_w,), jnp.int32),
            pltpu.VMEM((C, D), jnp.float32),
            pltpu.VMEM((C, D), jnp.float32),
            pltpu.SemaphoreType.DMA,
            pltpu.SemaphoreType.DMA,
            pltpu.SemaphoreType.DMA,
            pltpu.SemaphoreType.DMA,
        ],
    )
    def k(table_hbm, idx_hbm, out_hbm, idx_v, rows0, rows1, gs0, gs1, ws0, ws1):
        wid = lax.axis_index("s") * NC + lax.axis_index("c")
        base = wid * b_per_w
        pltpu.sync_copy(idx_hbm.at[pl.ds(base, b_per_w)], idx_v)

        def g_start(g, rows, sem):
            pltpu.async_copy(table_hbm.at[idx_v.at[pl.ds(g * C, C)]], rows, sem)

        def g_wait(rows, sem):
            pltpu.make_async_copy(
                table_hbm.at[idx_v.at[pl.ds(0, C)]], rows, sem
            ).wait()

        def w_start(g, rows, sem):
            pltpu.async_copy(rows, out_hbm.at[pl.ds(base + g * C, C)], sem)

        def w_wait(rows, sem):
            pltpu.make_async_copy(rows, out_hbm.at[pl.ds(base, C)], sem).wait()

        g_start(0, rows0, gs0)
        g_start(1, rows1, gs1)

        def body(j, carry):
            g0 = 2 * j
            g1 = g0 + 1
            g_wait(rows0, gs0)
            w_start(g0, rows0, ws0)
            g_wait(rows1, gs1)
            w_start(g1, rows1, ws1)
            w_wait(rows0, ws0)
            g_start(g0 + 2, rows0, gs0)
            w_wait(rows1, ws1)
            g_start(g1 + 2, rows1, gs1)
            return carry

        lax.fori_loop(0, n_chunks // 2 - 1, body, 0)

        g_wait(rows0, gs0)
        w_start(n_chunks - 2, rows0, ws0)
        g_wait(rows1, gs1)
        w_start(n_chunks - 1, rows1, ws1)
        w_wait(rows0, ws0)
        w_wait(rows1, ws1)

    return k(W, idx_flat)


def kernel(inp, W):
    B = inp.shape[0] * inp.shape[1]
    D = W.shape[1]
    idx_flat = inp.reshape(B).astype(jnp.int32)
    out = _gather_rows(idx_flat, W, B, D)
    return out.reshape(inp.shape[0], inp.shape[1], D)


# speedup vs baseline: 3.5917x; 1.3238x over previous
import functools

import jax
import jax.numpy as jnp
from jax import lax
from jax.experimental import pallas as pl
from jax.experimental.pallas import tpu as pltpu
from jax.experimental.pallas import tpu_sc as plsc

_CHUNK = 800


@functools.partial(jax.jit, static_argnames=("B", "D"))
def _gather_rows(idx_flat, W, B, D):
    info = plsc.get_sparse_core_info()
    NC, NS = info.num_cores, info.num_subcores
    NW = NC * NS
    b_per_w = B // NW
    n_chunks = b_per_w // _CHUNK
    C = _CHUNK
    mesh = plsc.VectorSubcoreMesh(core_axis_name="c", subcore_axis_name="s")

    @functools.partial(
        pl.kernel,
        mesh=mesh,
        compiler_params=pltpu.CompilerParams(use_tc_tiling_on_sc=False),
        out_type=jax.ShapeDtypeStruct((B, 128), jnp.float32),
        scratch_types=[
            pltpu.VMEM((b_per_w,), jnp.int32),
            pltpu.VMEM((C, D), jnp.float32),
            pltpu.VMEM((C, D), jnp.float32),
            pltpu.SemaphoreType.DMA,
            pltpu.SemaphoreType.DMA,
            pltpu.SemaphoreType.DMA,
            pltpu.SemaphoreType.DMA,
        ],
    )
    def k(table_hbm, idx_hbm, out_hbm, idx_v, rows0, rows1, gs0, gs1, ws0, ws1):
        wid = lax.axis_index("s") * NC + lax.axis_index("c")
        base = wid * b_per_w
        pltpu.sync_copy(idx_hbm.at[pl.ds(base, b_per_w)], idx_v)

        def g_start(g, rows, sem):
            pltpu.async_copy(table_hbm.at[idx_v.at[pl.ds(g * C, C)]], rows, sem)

        def g_wait(rows, sem):
            pltpu.make_async_copy(
                table_hbm.at[idx_v.at[pl.ds(0, C)]], rows, sem
            ).wait()

        def w_start(g, rows, sem):
            pltpu.async_copy(
                rows, out_hbm.at[pl.ds(base + g * C, C), pl.ds(0, D)], sem
            )

        def w_wait(rows, sem):
            pltpu.make_async_copy(
                rows, out_hbm.at[pl.ds(base, C), pl.ds(0, D)], sem
            ).wait()

        g_start(0, rows0, gs0)
        g_start(1, rows1, gs1)

        def body(j, carry):
            g0 = 2 * j
            g1 = g0 + 1
            g_wait(rows0, gs0)
            w_start(g0, rows0, ws0)
            g_wait(rows1, gs1)
            w_start(g1, rows1, ws1)
            w_wait(rows0, ws0)
            g_start(g0 + 2, rows0, gs0)
            w_wait(rows1, ws1)
            g_start(g1 + 2, rows1, gs1)
            return carry

        lax.fori_loop(0, n_chunks // 2 - 1, body, 0)

        g_wait(rows0, gs0)
        w_start(n_chunks - 2, rows0, ws0)
        g_wait(rows1, gs1)
        w_start(n_chunks - 1, rows1, ws1)
        w_wait(rows0, ws0)
        w_wait(rows1, ws1)

    return k(W, idx_flat)


def kernel(inp, W):
    B = inp.shape[0] * inp.shape[1]
    D = W.shape[1]
    idx_flat = inp.reshape(B).astype(jnp.int32)
    out = _gather_rows(idx_flat, W, B, D)
    return out.reshape(inp.shape[0], inp.shape[1], 128)[:, :, :D]
